# Initial kernel scaffold; baseline (speedup 1.0000x reference)
import functools

import jax
import jax.numpy as jnp
from jax import lax
from jax.experimental import pallas as pl
from jax.experimental.pallas import tpu as pltpu
from jax.experimental.pallas import tpu_sc as plsc

N = 10000
E = 320000
D = 128
H = 8
DK = 16
DFF = 4 * D
L = 2

NC = 2
NS = 16
NW = NC * NS
EPT = E // NW
CH = 80
NCHUNK = EPT // CH
RPT = N // NS
ZR = 125

BR = 1000


def _qkv_body(x_ref, wq_ref, bq_ref, wk_ref, wvw_ref, q_ref, k_ref, v_ref):
    x = x_ref[...]
    q = jnp.dot(x, wq_ref[...], preferred_element_type=jnp.float32)
    q_ref[...] = (q + bq_ref[...]) * 0.25
    k_ref[...] = jnp.dot(x, wk_ref[...], preferred_element_type=jnp.float32)
    v_ref[...] = jnp.dot(x, wvw_ref[...], preferred_element_type=jnp.float32)


_qkv_call = pl.pallas_call(
    _qkv_body,
    grid=(N // BR,),
    in_specs=[
        pl.BlockSpec((BR, D), lambda i: (i, 0)),
        pl.BlockSpec((D, D), lambda i: (0, 0)),
        pl.BlockSpec((1, D), lambda i: (0, 0)),
        pl.BlockSpec((D, D), lambda i: (0, 0)),
        pl.BlockSpec((D, D), lambda i: (0, 0)),
    ],
    out_specs=[pl.BlockSpec((BR, D), lambda i: (i, 0))] * 3,
    out_shape=[jax.ShapeDtypeStruct((N, D), jnp.float32)] * 3,
)


def _ln_rows(t, g, b):
    m = jnp.mean(t, axis=1, keepdims=True)
    c = t - m
    v = jnp.mean(c * c, axis=1, keepdims=True)
    return c * jax.lax.rsqrt(v + 1e-5) * g + b


def _post_body(x_ref, wvacc_ref, zacc_ref, wo_ref, bo_ref, g1_ref, be1_ref,
               w1_ref, b1_ref, w2_ref, b2_ref, g2_ref, be2_ref, out_ref):
    wv = wvacc_ref[0] + wvacc_ref[1]
    z = zacc_ref[0] + zacc_ref[1]
    rows = lax.broadcasted_iota(jnp.int32, (DK, D), 0)
    cols = lax.broadcasted_iota(jnp.int32, (DK, D), 1)
    expander = (cols // DK == rows).astype(jnp.float32)
    zfull = jnp.dot(z, expander, preferred_element_type=jnp.float32)
    o = wv / (zfull + 1e-9)
    y = x_ref[...] + jnp.dot(o, wo_ref[...], preferred_element_type=jnp.float32) + bo_ref[...]
    h = _ln_rows(y, g1_ref[...], be1_ref[...])
    ff = jnp.maximum(jnp.dot(h, w1_ref[...], preferred_element_type=jnp.float32) + b1_ref[...], 0.0)
    ff = jnp.dot(ff, w2_ref[...], preferred_element_type=jnp.float32) + b2_ref[...]
    out_ref[...] = _ln_rows(h + ff, g2_ref[...], be2_ref[...])


_post_call = pl.pallas_call(
    _post_body,
    grid=(N // BR,),
    in_specs=[
        pl.BlockSpec((BR, D), lambda i: (i, 0)),
        pl.BlockSpec((NC, BR, D), lambda i: (0, i, 0)),
        pl.BlockSpec((NC, BR, DK), lambda i: (0, i, 0)),
        pl.BlockSpec((D, D), lambda i: (0, 0)),
        pl.BlockSpec((1, D), lambda i: (0, 0)),
        pl.BlockSpec((1, D), lambda i: (0, 0)),
        pl.BlockSpec((1, D), lambda i: (0, 0)),
        pl.BlockSpec((D, DFF), lambda i: (0, 0)),
        pl.BlockSpec((1, DFF), lambda i: (0, 0)),
        pl.BlockSpec((DFF, D), lambda i: (0, 0)),
        pl.BlockSpec((1, D), lambda i: (0, 0)),
        pl.BlockSpec((1, D), lambda i: (0, 0)),
        pl.BlockSpec((1, D), lambda i: (0, 0)),
    ],
    out_specs=pl.BlockSpec((BR, D), lambda i: (i, 0)),
    out_shape=jax.ShapeDtypeStruct((N, D), jnp.float32),
)


@functools.partial(
    pl.kernel,
    out_type=(
        jax.ShapeDtypeStruct((NC, N, D), jnp.float32),
        jax.ShapeDtypeStruct((NC, N, DK), jnp.float32),
    ),
    mesh=plsc.VectorSubcoreMesh(core_axis_name="c", subcore_axis_name="s"),
    scratch_types=[
        pltpu.VMEM_SHARED((N, D), jnp.float32),
        pltpu.VMEM_SHARED((N, DK), jnp.float32),
        pltpu.VMEM((CH,), jnp.int32),
        pltpu.VMEM((CH,), jnp.int32),
        pltpu.VMEM((CH, D), jnp.float32),
        pltpu.VMEM((CH, D), jnp.float32),
        pltpu.VMEM((CH, D), jnp.float32),
        pltpu.VMEM((CH, D), jnp.float32),
        pltpu.VMEM((CH, DK), jnp.float32),
        pltpu.VMEM((ZR, D), jnp.float32),
        pltpu.VMEM((ZR, DK), jnp.float32),
        pltpu.SemaphoreType.DMA,
    ],
)
def _edge_kernel(q_hbm, k_hbm, v_hbm, ei_hbm, wv_out, z_out,
                 wv_sh, z_sh, src_v, dst_v, kk, qq, vv, wrows, zrows,
                 zb1, zb2, sem):
    cid = lax.axis_index("c")
    sid = lax.axis_index("s")
    wid = cid * NS + sid

    zeros16 = jnp.zeros((16,), jnp.float32)

    def zinit(i, carry):
        for j in range(D // 16):
            zb1[i, pl.ds(16 * j, 16)] = zeros16
        zb2[i, pl.ds(0, 16)] = zeros16
        return carry

    lax.fori_loop(0, ZR, zinit, 0)
    for rblk in range(RPT // ZR):
        off = sid * RPT + rblk * ZR
        pltpu.sync_copy(zb1, wv_sh.at[pl.ds(off, ZR)])
        pltpu.sync_copy(zb2, z_sh.at[pl.ds(off, ZR)])
    plsc.subcore_barrier()

    lane = lax.iota(jnp.int32, 16)

    def chunk_body(c, carry):
        base = wid * EPT + c * CH
        pltpu.sync_copy(ei_hbm.at[0, pl.ds(base, CH)], src_v)
        pltpu.sync_copy(ei_hbm.at[1, pl.ds(base, CH)], dst_v)
        cp1 = pltpu.async_copy(k_hbm.at[src_v], kk, sem)
        cp2 = pltpu.async_copy(q_hbm.at[dst_v], qq, sem)
        cp3 = pltpu.async_copy(v_hbm.at[src_v], vv, sem)
        cp1.wait()
        cp2.wait()
        cp3.wait()

        def edge_body(e, ecarry):
            zacc = jnp.zeros((16,), jnp.float32)
            for h in range(H):
                kv = kk[e, pl.ds(16 * h, 16)]
                qv = qq[e, pl.ds(16 * h, 16)]
                s = jnp.sum(kv * qv)
                es = jnp.exp(jnp.clip(jnp.broadcast_to(s, (16,)), -5.0, 5.0))
                wrows[e, pl.ds(16 * h, 16)] = vv[e, pl.ds(16 * h, 16)] * es
                zacc = zacc + jnp.where(lane == h, es, 0.0)
            zrows[e, pl.ds(0, 16)] = zacc
            return ecarry

        lax.fori_loop(0, CH, edge_body, 0)

        pltpu.sync_copy(wrows, wv_sh.at[dst_v], add=True)
        pltpu.sync_copy(zrows, z_sh.at[dst_v], add=True)
        return carry

    lax.fori_loop(0, NCHUNK, chunk_body, 0)

    plsc.subcore_barrier()
    off = sid * RPT
    pltpu.sync_copy(wv_sh.at[pl.ds(off, RPT)], wv_out.at[cid, pl.ds(off, RPT)])
    pltpu.sync_copy(z_sh.at[pl.ds(off, RPT)], z_out.at[cid, pl.ds(off, RPT)])


def kernel(x, edge_index, Wq, bq, Wk, Wv, Wo, bo, ln1_g, ln1_b, W1, b1, W2,
           b2, ln2_g, ln2_b):
    ei = edge_index
    for i in range(L):
        q, k, v = _qkv_call(x, Wq[i], bq[i].reshape(1, D), Wk[i], Wv[i])
        wv_acc, z_acc = _edge_kernel(q, k, v, ei)
        x = _post_call(x, wv_acc, z_acc, Wo[i], bo[i].reshape(1, D),
                       ln1_g[i].reshape(1, D), ln1_b[i].reshape(1, D),
                       W1[i], b1[i].reshape(1, DFF), W2[i],
                       b2[i].reshape(1, D), ln2_g[i].reshape(1, D),
                       ln2_b[i].reshape(1, D))
    return x


# speedup vs baseline: 12.5171x; 12.5171x over previous
import functools

import jax
import jax.numpy as jnp
from jax import lax
from jax.experimental import pallas as pl
from jax.experimental.pallas import tpu as pltpu
from jax.experimental.pallas import tpu_sc as plsc

N = 10000
E = 320000
D = 128
H = 8
DK = 16
DH = D // 2
HH = H // 2
DFF = 4 * D
L = 2

NC = 2
NS = 16
EPT = E // NS
CH = 32
NCHUNK = EPT // CH
NP_ = 10240
WVR = NP_ // 2
ZR_ = NP_ // 8
WVT = WVR // NS
ZT_ = ZR_ // NS

BR = 1000


def _qkv_body(x_ref, wq_ref, bq_ref, wk0_ref, wk1_ref, wv0_ref,
              wv1_ref, q_ref, kv_ref):
    x = x_ref[...]
    q_ref[...] = (jnp.dot(x, wq_ref[...], preferred_element_type=jnp.float32)
                  + bq_ref[...]) * 0.25
    kv_ref[0] = jnp.concatenate(
        [jnp.dot(x, wk0_ref[...], preferred_element_type=jnp.float32),
         jnp.dot(x, wv0_ref[...], preferred_element_type=jnp.float32)], axis=1)
    kv_ref[1] = jnp.concatenate(
        [jnp.dot(x, wk1_ref[...], preferred_element_type=jnp.float32),
         jnp.dot(x, wv1_ref[...], preferred_element_type=jnp.float32)], axis=1)


_qkv_call = pl.pallas_call(
    _qkv_body,
    grid=(N // BR,),
    in_specs=[
        pl.BlockSpec((BR, D), lambda i: (i, 0)),
        pl.BlockSpec((D, D), lambda i: (0, 0)),
        pl.BlockSpec((1, D), lambda i: (0, 0)),
    ]
    + [pl.BlockSpec((D, DH), lambda i: (0, 0))] * 4,
    out_specs=[pl.BlockSpec((BR, D), lambda i: (i, 0)),
               pl.BlockSpec((NC, BR, D), lambda i: (0, i, 0))],
    out_shape=[jax.ShapeDtypeStruct((N, D), jnp.float32),
               jax.ShapeDtypeStruct((NC, N, D), jnp.float32)],
)


def _ln_rows(t, g, b):
    m = jnp.mean(t, axis=1, keepdims=True)
    c = t - m
    v = jnp.mean(c * c, axis=1, keepdims=True)
    return c * jax.lax.rsqrt(v + 1e-5) * g + b


def _post_body(x_ref, wvacc_ref, zacc_ref, wo0_ref, wo1_ref, bo_ref, g1_ref,
               be1_ref, w1_ref, b1_ref, w2_ref, b2_ref, g2_ref, be2_ref,
               out_ref):
    z = zacc_ref[0] + zacc_ref[1]
    rows = lax.broadcasted_iota(jnp.int32, (DK, DH), 0)
    cols = lax.broadcasted_iota(jnp.int32, (DK, DH), 1)
    exp0 = (cols // DK == rows).astype(jnp.float32)
    exp1 = ((cols + DH) // DK == rows).astype(jnp.float32)
    o0 = wvacc_ref[0] / (jnp.dot(z, exp0, preferred_element_type=jnp.float32)
                         + 1e-9)
    o1 = wvacc_ref[1] / (jnp.dot(z, exp1, preferred_element_type=jnp.float32)
                         + 1e-9)
    y = (x_ref[...] + bo_ref[...]
         + jnp.dot(o0, wo0_ref[...], preferred_element_type=jnp.float32)
         + jnp.dot(o1, wo1_ref[...], preferred_element_type=jnp.float32))
    h = _ln_rows(y, g1_ref[...], be1_ref[...])
    ff = jnp.maximum(jnp.dot(h, w1_ref[...], preferred_element_type=jnp.float32) + b1_ref[...], 0.0)
    ff = jnp.dot(ff, w2_ref[...], preferred_element_type=jnp.float32) + b2_ref[...]
    out_ref[...] = _ln_rows(h + ff, g2_ref[...], be2_ref[...])


_post_call = pl.pallas_call(
    _post_body,
    grid=(N // BR,),
    in_specs=[
        pl.BlockSpec((BR, D), lambda i: (i, 0)),
        pl.BlockSpec((NC, BR, DH), lambda i: (0, i, 0)),
        pl.BlockSpec((NC, BR, DK), lambda i: (0, i, 0)),
        pl.BlockSpec((DH, D), lambda i: (0, 0)),
        pl.BlockSpec((DH, D), lambda i: (0, 0)),
        pl.BlockSpec((1, D), lambda i: (0, 0)),
        pl.BlockSpec((1, D), lambda i: (0, 0)),
        pl.BlockSpec((1, D), lambda i: (0, 0)),
        pl.BlockSpec((D, DFF), lambda i: (0, 0)),
        pl.BlockSpec((1, DFF), lambda i: (0, 0)),
        pl.BlockSpec((DFF, D), lambda i: (0, 0)),
        pl.BlockSpec((1, D), lambda i: (0, 0)),
        pl.BlockSpec((1, D), lambda i: (0, 0)),
        pl.BlockSpec((1, D), lambda i: (0, 0)),
    ],
    out_specs=pl.BlockSpec((BR, D), lambda i: (i, 0)),
    out_shape=jax.ShapeDtypeStruct((N, D), jnp.float32),
)


@functools.partial(
    pl.kernel,
    out_type=(
        jax.ShapeDtypeStruct((NC, WVR, D), jnp.float32),
        jax.ShapeDtypeStruct((NC, ZR_, D), jnp.float32),
    ),
    mesh=plsc.VectorSubcoreMesh(core_axis_name="c", subcore_axis_name="s"),
    compiler_params=pltpu.CompilerParams(needs_layout_passes=False),
    scratch_types=[
        pltpu.VMEM_SHARED((WVR, D), jnp.float32),
        pltpu.VMEM_SHARED((ZR_, D), jnp.float32),
        pltpu.VMEM((CH,), jnp.int32),
        pltpu.VMEM((CH,), jnp.int32),
        pltpu.VMEM((CH,), jnp.int32),
        pltpu.VMEM((CH,), jnp.int32),
        pltpu.VMEM((CH,), jnp.int32),
        pltpu.VMEM((CH, D), jnp.float32),
        pltpu.VMEM((CH, D), jnp.float32),
        pltpu.VMEM((CH, D), jnp.float32),
        pltpu.VMEM((CH, D), jnp.float32),
        pltpu.SemaphoreType.DMA,
    ],
)
def _edge_kernel(q_hbm, kv_hbm, src_hbm,
                 dst_hbm, wv_out, z_out, wv_sh, z_sh, src_v, dst_v, src2_v,
                 dst2_v, dst8_v, kk, qq, wrows, zrows, sem):
    cid = lax.axis_index("c")
    sid = lax.axis_index("s")

    zeros16 = jnp.zeros((16,), jnp.float32)

    def binit(i, carry):
        for j in range(D // 16):
            wrows[i, pl.ds(16 * j, 16)] = zeros16
        return carry

    lax.fori_loop(0, CH, binit, 0)
    for rblk in range(WVT // CH):
        off = sid * WVT + rblk * CH
        pltpu.sync_copy(wrows, wv_sh.at[pl.ds(off, CH)])
    for rblk in range(ZT_ // CH):
        off = sid * ZT_ + rblk * CH
        pltpu.sync_copy(wrows, z_sh.at[pl.ds(off, CH)])
    zoff = sid * ZT_ + (ZT_ // CH) * CH
    pltpu.sync_copy(wrows.at[pl.ds(0, ZT_ % CH)], z_sh.at[pl.ds(zoff, ZT_ % CH)])
    plsc.subcore_barrier()

    lane = lax.iota(jnp.int32, 16)
    hbase = cid * HH
    qbase = cid * DH

    def chunk_body(c, carry):
        base = sid * EPT + c * CH
        pltpu.sync_copy(src_hbm.at[pl.ds(base, CH)], src_v)
        pltpu.sync_copy(dst_hbm.at[pl.ds(base, CH)], dst_v)

        def sh_body(j, carry):
            src2_v[pl.ds(16 * j, 16)] = src_v[pl.ds(16 * j, 16)] + cid * N
            dvec = dst_v[pl.ds(16 * j, 16)]
            dst2_v[pl.ds(16 * j, 16)] = dvec >> 1
            dst8_v[pl.ds(16 * j, 16)] = dvec >> 3
            return carry

        lax.fori_loop(0, CH // 16, sh_body, 0)
        cpq = pltpu.async_copy(q_hbm.at[dst_v], qq, sem)
        cpk = pltpu.async_copy(kv_hbm.at[src2_v], kk, sem)
        cpq.wait()
        cpk.wait()

        def group_body(g, gcarry):
            dvec = dst_v[pl.ds(16 * g, 16)]
            for i in range(16):
                e = 16 * g + i
                d = dvec[i]
                par = (d & 1) * DH
                opp = DH - par
                slot = (d & 7) * DK
                zacc = jnp.zeros((16,), jnp.float32)
                for h in range(HH):
                    kv = kk[e, pl.ds(16 * h, 16)]
                    qv = qq[e, pl.ds(qbase + 16 * h, 16)]
                    cs = plsc.cumsum(kv * qv)
                    sb = jnp.broadcast_to(cs[15], (16,))
                    es = jnp.exp(jnp.clip(sb, -5.0, 5.0))
                    wrows[e, pl.ds(par + 16 * h, 16)] = (
                        kk[e, pl.ds(DH + 16 * h, 16)] * es)
                    wrows[e, pl.ds(opp + 16 * h, 16)] = zeros16
                    zacc = zacc + jnp.where(lane == hbase + h, es, 0.0)
                for j in range(8):
                    zrows[e, pl.ds(16 * j, 16)] = jnp.where(
                        jnp.broadcast_to(slot == 16 * j, (16,)), zacc, zeros16)
            return gcarry

        lax.fori_loop(0, CH // 16, group_body, 0)

        pltpu.sync_copy(wrows, wv_sh.at[dst2_v], add=True)
        pltpu.sync_copy(zrows, z_sh.at[dst8_v], add=True)
        return carry

    lax.fori_loop(0, NCHUNK, chunk_body, 0)

    plsc.subcore_barrier()
    woff = sid * WVT
    pltpu.sync_copy(wv_sh.at[pl.ds(woff, WVT)],
                    wv_out.at[cid, pl.ds(woff, WVT)])
    zoff2 = sid * ZT_
    pltpu.sync_copy(z_sh.at[pl.ds(zoff2, ZT_)],
                    z_out.at[cid, pl.ds(zoff2, ZT_)])


def kernel(x, edge_index, Wq, bq, Wk, Wv, Wo, bo, ln1_g, ln1_b, W1, b1, W2,
           b2, ln2_g, ln2_b):
    src = edge_index[0]
    dst = edge_index[1]
    for i in range(L):
        q, kv = _qkv_call(
            x, Wq[i], bq[i].reshape(1, D),
            Wk[i, :, :DH], Wk[i, :, DH:], Wv[i, :, :DH], Wv[i, :, DH:])
        wv_acc, z_acc = _edge_kernel(q, kv.reshape(NC * N, D), src, dst)
        wv_acc = wv_acc.reshape(NC, NP_, DH)
        z_acc = z_acc.reshape(NC, NP_, DK)
        x = _post_call(x, wv_acc, z_acc, Wo[i, :DH, :], Wo[i, DH:, :],
                       bo[i].reshape(1, D),
                       ln1_g[i].reshape(1, D), ln1_b[i].reshape(1, D),
                       W1[i], b1[i].reshape(1, DFF), W2[i],
                       b2[i].reshape(1, D), ln2_g[i].reshape(1, D),
                       ln2_b[i].reshape(1, D))
    return x


# speedup vs baseline: 16.2563x; 1.2987x over previous
import functools

import jax
import jax.numpy as jnp
from jax import lax
from jax.experimental import pallas as pl
from jax.experimental.pallas import tpu as pltpu
from jax.experimental.pallas import tpu_sc as plsc

N = 10000
E = 320000
D = 128
H = 8
DK = 16
DH = D // 2
HH = H // 2
DFF = 4 * D
L = 2

NC = 2
NS = 16
EPT = E // NS
CH = 32
NCHUNK = EPT // CH
NP_ = 10240
WVR = NP_ // 2
ZR_ = NP_ // 8
WVT = WVR // NS
ZT_ = ZR_ // NS

BR = 1000


def _qkv_body(x_ref, wq_ref, bq_ref, wk0_ref, wk1_ref, wv0_ref,
              wv1_ref, q_ref, kv_ref):
    x = x_ref[...]
    q_ref[...] = (jnp.dot(x, wq_ref[...], preferred_element_type=jnp.float32)
                  + bq_ref[...]) * 0.25
    kv_ref[0] = jnp.concatenate(
        [jnp.dot(x, wk0_ref[...], preferred_element_type=jnp.float32),
         jnp.dot(x, wv0_ref[...], preferred_element_type=jnp.float32)], axis=1)
    kv_ref[1] = jnp.concatenate(
        [jnp.dot(x, wk1_ref[...], preferred_element_type=jnp.float32),
         jnp.dot(x, wv1_ref[...], preferred_element_type=jnp.float32)], axis=1)


_qkv_call = pl.pallas_call(
    _qkv_body,
    grid=(N // BR,),
    in_specs=[
        pl.BlockSpec((BR, D), lambda i: (i, 0)),
        pl.BlockSpec((D, D), lambda i: (0, 0)),
        pl.BlockSpec((1, D), lambda i: (0, 0)),
    ]
    + [pl.BlockSpec((D, DH), lambda i: (0, 0))] * 4,
    out_specs=[pl.BlockSpec((BR, D), lambda i: (i, 0)),
               pl.BlockSpec((NC, BR, D), lambda i: (0, i, 0))],
    out_shape=[jax.ShapeDtypeStruct((N, D), jnp.float32),
               jax.ShapeDtypeStruct((NC, N, D), jnp.float32)],
)


def _ln_rows(t, g, b):
    m = jnp.mean(t, axis=1, keepdims=True)
    c = t - m
    v = jnp.mean(c * c, axis=1, keepdims=True)
    return c * jax.lax.rsqrt(v + 1e-5) * g + b


def _post_body(x_ref, wvacc_ref, zacc_ref, wo0_ref, wo1_ref, bo_ref, g1_ref,
               be1_ref, w1_ref, b1_ref, w2_ref, b2_ref, g2_ref, be2_ref,
               out_ref):
    z = zacc_ref[0] + zacc_ref[1]
    rows = lax.broadcasted_iota(jnp.int32, (DK, DH), 0)
    cols = lax.broadcasted_iota(jnp.int32, (DK, DH), 1)
    exp0 = (cols // DK == rows).astype(jnp.float32)
    exp1 = ((cols + DH) // DK == rows).astype(jnp.float32)
    o0 = wvacc_ref[0] / (jnp.dot(z, exp0, preferred_element_type=jnp.float32)
                         + 1e-9)
    o1 = wvacc_ref[1] / (jnp.dot(z, exp1, preferred_element_type=jnp.float32)
                         + 1e-9)
    y = (x_ref[...] + bo_ref[...]
         + jnp.dot(o0, wo0_ref[...], preferred_element_type=jnp.float32)
         + jnp.dot(o1, wo1_ref[...], preferred_element_type=jnp.float32))
    h = _ln_rows(y, g1_ref[...], be1_ref[...])
    ff = jnp.maximum(jnp.dot(h, w1_ref[...], preferred_element_type=jnp.float32) + b1_ref[...], 0.0)
    ff = jnp.dot(ff, w2_ref[...], preferred_element_type=jnp.float32) + b2_ref[...]
    out_ref[...] = _ln_rows(h + ff, g2_ref[...], be2_ref[...])


_post_call = pl.pallas_call(
    _post_body,
    grid=(N // BR,),
    in_specs=[
        pl.BlockSpec((BR, D), lambda i: (i, 0)),
        pl.BlockSpec((NC, BR, DH), lambda i: (0, i, 0)),
        pl.BlockSpec((NC, BR, DK), lambda i: (0, i, 0)),
        pl.BlockSpec((DH, D), lambda i: (0, 0)),
        pl.BlockSpec((DH, D), lambda i: (0, 0)),
        pl.BlockSpec((1, D), lambda i: (0, 0)),
        pl.BlockSpec((1, D), lambda i: (0, 0)),
        pl.BlockSpec((1, D), lambda i: (0, 0)),
        pl.BlockSpec((D, DFF), lambda i: (0, 0)),
        pl.BlockSpec((1, DFF), lambda i: (0, 0)),
        pl.BlockSpec((DFF, D), lambda i: (0, 0)),
        pl.BlockSpec((1, D), lambda i: (0, 0)),
        pl.BlockSpec((1, D), lambda i: (0, 0)),
        pl.BlockSpec((1, D), lambda i: (0, 0)),
    ],
    out_specs=pl.BlockSpec((BR, D), lambda i: (i, 0)),
    out_shape=jax.ShapeDtypeStruct((N, D), jnp.float32),
)


@functools.partial(
    pl.kernel,
    out_type=(
        jax.ShapeDtypeStruct((NC, WVR, D), jnp.float32),
        jax.ShapeDtypeStruct((NC, ZR_, D), jnp.float32),
    ),
    mesh=plsc.VectorSubcoreMesh(core_axis_name="c", subcore_axis_name="s"),
    compiler_params=pltpu.CompilerParams(needs_layout_passes=False),
    scratch_types=[
        pltpu.VMEM_SHARED((WVR, D), jnp.float32),
        pltpu.VMEM_SHARED((ZR_, D), jnp.float32),
        pltpu.VMEM((2 * CH,), jnp.int32),
        pltpu.VMEM((2 * CH,), jnp.int32),
        pltpu.VMEM((CH,), jnp.int32),
        pltpu.VMEM((CH,), jnp.int32),
        pltpu.VMEM((CH,), jnp.int32),
        pltpu.VMEM((CH,), jnp.int32),
        pltpu.VMEM((CH,), jnp.int32),
        pltpu.VMEM((CH,), jnp.int32),
        pltpu.VMEM((CH,), jnp.int32),
        pltpu.VMEM((CH,), jnp.int32),
        pltpu.VMEM((CH, D), jnp.float32),
        pltpu.VMEM((CH, D), jnp.float32),
        pltpu.VMEM((CH, D), jnp.float32),
        pltpu.VMEM((CH, D), jnp.float32),
        pltpu.VMEM((CH, D), jnp.float32),
        pltpu.VMEM((CH, D), jnp.float32),
        pltpu.VMEM((CH, D), jnp.float32),
        pltpu.VMEM((CH, D), jnp.float32),
        pltpu.SemaphoreType.DMA,
        pltpu.SemaphoreType.DMA,
        pltpu.SemaphoreType.DMA,
        pltpu.SemaphoreType.DMA,
        pltpu.SemaphoreType.DMA,
    ],
)
def _edge_kernel(q_hbm, kv_hbm, src_hbm, dst_hbm, wv_out, z_out, wv_sh, z_sh,
                 stg_s, stg_d, s2A, qdA, d2A, d8A, s2B, qdB, d2B, d8B,
                 kkA, qqA, wrA, zrA, kkB, qqB, wrB, zrB,
                 isem, gsA, gsB, ssA, ssB):
    cid = lax.axis_index("c")
    sid = lax.axis_index("s")

    zeros16 = jnp.zeros((16,), jnp.float32)

    def binit(i, carry):
        for j in range(D // 16):
            wrA[i, pl.ds(16 * j, 16)] = zeros16
        return carry

    lax.fori_loop(0, CH, binit, 0)
    for rblk in range(WVT // CH):
        off = sid * WVT + rblk * CH
        pltpu.sync_copy(wrA, wv_sh.at[pl.ds(off, CH)])
    for rblk in range(ZT_ // CH):
        off = sid * ZT_ + rblk * CH
        pltpu.sync_copy(wrA, z_sh.at[pl.ds(off, CH)])
    zoff = sid * ZT_ + (ZT_ // CH) * CH
    pltpu.sync_copy(wrA.at[pl.ds(0, ZT_ % CH)], z_sh.at[pl.ds(zoff, ZT_ % CH)])
    plsc.subcore_barrier()

    lane = lax.iota(jnp.int32, 16)
    hbase = cid * HH
    qbase = cid * DH

    ebase = sid * EPT
    NU = (NCHUNK - 1) // 2

    def stage_block(first_chunk):
        off = ebase + first_chunk * CH
        pltpu.async_copy(src_hbm.at[pl.ds(off, 2 * CH)], stg_s, isem)
        pltpu.async_copy(dst_hbm.at[pl.ds(off, 2 * CH)], stg_d, isem)

    def wait_stage():
        pltpu.make_async_copy(src_hbm.at[pl.ds(0, 2 * CH)], stg_s, isem).wait()
        pltpu.make_async_copy(dst_hbm.at[pl.ds(0, 2 * CH)], stg_d, isem).wait()

    def derive_pre(stg_off, s2, qd):
        for j in range(CH // 16):
            sv = stg_s[pl.ds(stg_off + 16 * j, 16)]
            dv = stg_d[pl.ds(stg_off + 16 * j, 16)]
            s2[pl.ds(16 * j, 16)] = sv + cid * N
            qd[pl.ds(16 * j, 16)] = dv

    def derive_post(qd, d2, d8):
        for j in range(CH // 16):
            dv = qd[pl.ds(16 * j, 16)]
            d2[pl.ds(16 * j, 16)] = dv >> 1
            d8[pl.ds(16 * j, 16)] = dv >> 3

    def issue_gathers(s2, qd, kkX, qqX, sem):
        pltpu.async_copy(kv_hbm.at[s2], kkX, sem)
        pltpu.async_copy(q_hbm.at[qd], qqX, sem)

    def wait_gathers(s2, qd, kkX, qqX, sem):
        pltpu.make_async_copy(kv_hbm.at[s2], kkX, sem).wait()
        pltpu.make_async_copy(q_hbm.at[qd], qqX, sem).wait()

    def issue_scatters(wrX, zrX, d2, d8, sem):
        pltpu.async_copy(wrX, wv_sh.at[d2], sem, add=True)
        pltpu.async_copy(zrX, z_sh.at[d8], sem, add=True)

    def wait_scatters(wrX, zrX, d2, d8, sem):
        pltpu.make_async_copy(wrX, wv_sh.at[d2], sem).wait()
        pltpu.make_async_copy(zrX, z_sh.at[d8], sem).wait()

    def compute(kkX, qqX, wrX, zrX, qd):
        def group_body(g, gcarry):
            dvec = qd[pl.ds(16 * g, 16)]
            for i in range(16):
                e = 16 * g + i
                d = dvec[i]
                par = (d & 1) * DH
                opp = DH - par
                slot = (d & 7) * DK
                zacc = jnp.zeros((16,), jnp.float32)
                for h in range(HH):
                    kv = kkX[e, pl.ds(16 * h, 16)]
                    qv = qqX[e, pl.ds(qbase + 16 * h, 16)]
                    cs = plsc.cumsum(kv * qv)
                    sb = jnp.broadcast_to(cs[15], (16,))
                    es = jnp.exp(jnp.clip(sb, -5.0, 5.0))
                    wrX[e, pl.ds(par + 16 * h, 16)] = (
                        kkX[e, pl.ds(DH + 16 * h, 16)] * es)
                    wrX[e, pl.ds(opp + 16 * h, 16)] = zeros16
                    zacc = zacc + jnp.where(lane == hbase + h, es, 0.0)
                for j in range(8):
                    zrX[e, pl.ds(16 * j, 16)] = jnp.where(
                        jnp.broadcast_to(slot == 16 * j, (16,)), zacc, zeros16)
            return gcarry

        lax.fori_loop(0, CH // 16, group_body, 0)

    pltpu.sync_copy(src_hbm.at[pl.ds(ebase, CH)], stg_s.at[pl.ds(0, CH)])
    pltpu.sync_copy(dst_hbm.at[pl.ds(ebase, CH)], stg_d.at[pl.ds(0, CH)])
    derive_pre(0, s2A, qdA)
    issue_gathers(s2A, qdA, kkA, qqA, gsA)
    stage_block(1)

    def pipe_body(u, carry):
        wait_stage()
        derive_pre(0, s2B, qdB)
        issue_gathers(s2B, qdB, kkB, qqB, gsB)
        wait_gathers(s2A, qdA, kkA, qqA, gsA)

        @pl.when(u > 0)
        def _():
            wait_scatters(wrA, zrA, d2A, d8A, ssA)

        derive_post(qdA, d2A, d8A)
        compute(kkA, qqA, wrA, zrA, qdA)
        issue_scatters(wrA, zrA, d2A, d8A, ssA)

        derive_pre(CH, s2A, qdA)
        issue_gathers(s2A, qdA, kkA, qqA, gsA)

        @pl.when(u < NU - 1)
        def _():
            stage_block(2 * u + 3)

        wait_gathers(s2B, qdB, kkB, qqB, gsB)

        @pl.when(u > 0)
        def _():
            wait_scatters(wrB, zrB, d2B, d8B, ssB)

        derive_post(qdB, d2B, d8B)
        compute(kkB, qqB, wrB, zrB, qdB)
        issue_scatters(wrB, zrB, d2B, d8B, ssB)
        return carry

    lax.fori_loop(0, NU, pipe_body, 0)

    wait_gathers(s2A, qdA, kkA, qqA, gsA)
    wait_scatters(wrA, zrA, d2A, d8A, ssA)
    derive_post(qdA, d2A, d8A)
    compute(kkA, qqA, wrA, zrA, qdA)
    issue_scatters(wrA, zrA, d2A, d8A, ssA)
    wait_scatters(wrA, zrA, d2A, d8A, ssA)
    wait_scatters(wrB, zrB, d2B, d8B, ssB)

    plsc.subcore_barrier()
    woff = sid * WVT
    pltpu.sync_copy(wv_sh.at[pl.ds(woff, WVT)],
                    wv_out.at[cid, pl.ds(woff, WVT)])
    zoff2 = sid * ZT_
    pltpu.sync_copy(z_sh.at[pl.ds(zoff2, ZT_)],
                    z_out.at[cid, pl.ds(zoff2, ZT_)])


def kernel(x, edge_index, Wq, bq, Wk, Wv, Wo, bo, ln1_g, ln1_b, W1, b1, W2,
           b2, ln2_g, ln2_b):
    src = edge_index[0]
    dst = edge_index[1]
    for i in range(L):
        q, kv = _qkv_call(
            x, Wq[i], bq[i].reshape(1, D),
            Wk[i, :, :DH], Wk[i, :, DH:], Wv[i, :, :DH], Wv[i, :, DH:])
        wv_acc, z_acc = _edge_kernel(q, kv.reshape(NC * N, D), src, dst)
        wv_acc = wv_acc.reshape(NC, NP_, DH)
        z_acc = z_acc.reshape(NC, NP_, DK)
        x = _post_call(x, wv_acc, z_acc, Wo[i, :DH, :], Wo[i, DH:, :],
                       bo[i].reshape(1, D),
                       ln1_g[i].reshape(1, D), ln1_b[i].reshape(1, D),
                       W1[i], b1[i].reshape(1, DFF), W2[i],
                       b2[i].reshape(1, D), ln2_g[i].reshape(1, D),
                       ln2_b[i].reshape(1, D))
    return x


# speedup vs baseline: 19.3609x; 1.1910x over previous
import functools

import jax
import jax.numpy as jnp
from jax import lax
from jax.experimental import pallas as pl
from jax.experimental.pallas import tpu as pltpu
from jax.experimental.pallas import tpu_sc as plsc

N = 10000
E = 320000
D = 128
H = 8
DK = 16
DH = D // 2
HH = H // 2
DFF = 4 * D
L = 2

NC = 2
NS = 16
EPT = E // NS
CH = 32
NCHUNK = EPT // CH
NP_ = 10240
WVR = NP_ // 2
ZR_ = NP_ // 8
WVT = WVR // NS
ZT_ = ZR_ // NS

BR = 1000


def _qkv_body(x_ref, wq_ref, bq_ref, wk0_ref, wk1_ref, wv0_ref,
              wv1_ref, q_ref, kv_ref):
    x = x_ref[...]
    q_ref[...] = (jnp.dot(x, wq_ref[...], preferred_element_type=jnp.float32)
                  + bq_ref[...]) * 0.25
    kv_ref[0] = jnp.concatenate(
        [jnp.dot(x, wk0_ref[...], preferred_element_type=jnp.float32),
         jnp.dot(x, wv0_ref[...], preferred_element_type=jnp.float32)], axis=1)
    kv_ref[1] = jnp.concatenate(
        [jnp.dot(x, wk1_ref[...], preferred_element_type=jnp.float32),
         jnp.dot(x, wv1_ref[...], preferred_element_type=jnp.float32)], axis=1)


_qkv_call = pl.pallas_call(
    _qkv_body,
    grid=(N // BR,),
    in_specs=[
        pl.BlockSpec((BR, D), lambda i: (i, 0)),
        pl.BlockSpec((D, D), lambda i: (0, 0)),
        pl.BlockSpec((1, D), lambda i: (0, 0)),
    ]
    + [pl.BlockSpec((D, DH), lambda i: (0, 0))] * 4,
    out_specs=[pl.BlockSpec((BR, D), lambda i: (i, 0)),
               pl.BlockSpec((NC, BR, D), lambda i: (0, i, 0))],
    out_shape=[jax.ShapeDtypeStruct((N, D), jnp.float32),
               jax.ShapeDtypeStruct((NC, N, D), jnp.float32)],
)


def _ln_rows(t, g, b):
    m = jnp.mean(t, axis=1, keepdims=True)
    c = t - m
    v = jnp.mean(c * c, axis=1, keepdims=True)
    return c * jax.lax.rsqrt(v + 1e-5) * g + b


def _post_body(x_ref, wvacc_ref, zacc_ref, wo0_ref, wo1_ref, bo_ref, g1_ref,
               be1_ref, w1_ref, b1_ref, w2_ref, b2_ref, g2_ref, be2_ref,
               out_ref):
    z = zacc_ref[0] + zacc_ref[1]
    rows = lax.broadcasted_iota(jnp.int32, (DK, DH), 0)
    cols = lax.broadcasted_iota(jnp.int32, (DK, DH), 1)
    exp0 = (cols // DK == rows).astype(jnp.float32)
    exp1 = ((cols + DH) // DK == rows).astype(jnp.float32)
    o0 = wvacc_ref[0] / (jnp.dot(z, exp0, preferred_element_type=jnp.float32)
                         + 1e-9)
    o1 = wvacc_ref[1] / (jnp.dot(z, exp1, preferred_element_type=jnp.float32)
                         + 1e-9)
    y = (x_ref[...] + bo_ref[...]
         + jnp.dot(o0, wo0_ref[...], preferred_element_type=jnp.float32)
         + jnp.dot(o1, wo1_ref[...], preferred_element_type=jnp.float32))
    h = _ln_rows(y, g1_ref[...], be1_ref[...])
    ff = jnp.maximum(jnp.dot(h, w1_ref[...], preferred_element_type=jnp.float32) + b1_ref[...], 0.0)
    ff = jnp.dot(ff, w2_ref[...], preferred_element_type=jnp.float32) + b2_ref[...]
    out_ref[...] = _ln_rows(h + ff, g2_ref[...], be2_ref[...])


_post_call = pl.pallas_call(
    _post_body,
    grid=(N // BR,),
    in_specs=[
        pl.BlockSpec((BR, D), lambda i: (i, 0)),
        pl.BlockSpec((NC, BR, DH), lambda i: (0, i, 0)),
        pl.BlockSpec((NC, BR, DK), lambda i: (0, i, 0)),
        pl.BlockSpec((DH, D), lambda i: (0, 0)),
        pl.BlockSpec((DH, D), lambda i: (0, 0)),
        pl.BlockSpec((1, D), lambda i: (0, 0)),
        pl.BlockSpec((1, D), lambda i: (0, 0)),
        pl.BlockSpec((1, D), lambda i: (0, 0)),
        pl.BlockSpec((D, DFF), lambda i: (0, 0)),
        pl.BlockSpec((1, DFF), lambda i: (0, 0)),
        pl.BlockSpec((DFF, D), lambda i: (0, 0)),
        pl.BlockSpec((1, D), lambda i: (0, 0)),
        pl.BlockSpec((1, D), lambda i: (0, 0)),
        pl.BlockSpec((1, D), lambda i: (0, 0)),
    ],
    out_specs=pl.BlockSpec((BR, D), lambda i: (i, 0)),
    out_shape=jax.ShapeDtypeStruct((N, D), jnp.float32),
)


@functools.partial(
    pl.kernel,
    out_type=(
        jax.ShapeDtypeStruct((NC, WVR, D), jnp.float32),
        jax.ShapeDtypeStruct((NC, ZR_, D), jnp.float32),
    ),
    mesh=plsc.VectorSubcoreMesh(core_axis_name="c", subcore_axis_name="s"),
    compiler_params=pltpu.CompilerParams(needs_layout_passes=False),
    scratch_types=[
        pltpu.VMEM_SHARED((WVR, D), jnp.float32),
        pltpu.VMEM_SHARED((ZR_, D), jnp.float32),
        pltpu.VMEM((2 * CH,), jnp.int32),
        pltpu.VMEM((2 * CH,), jnp.int32),
        pltpu.VMEM((CH,), jnp.int32),
        pltpu.VMEM((CH,), jnp.int32),
        pltpu.VMEM((CH,), jnp.int32),
        pltpu.VMEM((CH,), jnp.int32),
        pltpu.VMEM((CH,), jnp.int32),
        pltpu.VMEM((CH,), jnp.int32),
        pltpu.VMEM((CH,), jnp.int32),
        pltpu.VMEM((CH,), jnp.int32),
        pltpu.VMEM((CH,), jnp.int32),
        pltpu.VMEM((CH,), jnp.int32),
        pltpu.VMEM((CH, D), jnp.float32),
        pltpu.VMEM((CH, D), jnp.float32),
        pltpu.VMEM((CH, D), jnp.float32),
        pltpu.VMEM((CH, D), jnp.float32),
        pltpu.VMEM((CH, D), jnp.float32),
        pltpu.VMEM((CH, D), jnp.float32),
        pltpu.VMEM((CH, D), jnp.float32),
        pltpu.VMEM((CH, D), jnp.float32),
        pltpu.SemaphoreType.DMA,
        pltpu.SemaphoreType.DMA,
        pltpu.SemaphoreType.DMA,
        pltpu.SemaphoreType.DMA,
        pltpu.SemaphoreType.DMA,
    ],
)
def _edge_kernel(q_hbm, kv_hbm, src_hbm, dst_hbm, wv_out, z_out, wv_sh, z_sh,
                 stg_s, stg_d, s2A, qdA, d2A, d8A, s2B, qdB, d2B, d8B,
                 psA, psB, kkA, qqA, wrA, zrA, kkB, qqB, wrB, zrB,
                 isem, gsA, gsB, ssA, ssB):
    cid = lax.axis_index("c")
    sid = lax.axis_index("s")

    zeros16 = jnp.zeros((16,), jnp.float32)

    def binit(i, carry):
        for j in range(D // 16):
            wrA[i, pl.ds(16 * j, 16)] = zeros16
            zrA[i, pl.ds(16 * j, 16)] = zeros16
            zrB[i, pl.ds(16 * j, 16)] = zeros16
        return carry

    lax.fori_loop(0, CH, binit, 0)
    for j in range(CH // 16):
        psA[pl.ds(16 * j, 16)] = lax.iota(jnp.int32, 16) * 0
        psB[pl.ds(16 * j, 16)] = lax.iota(jnp.int32, 16) * 0
    for rblk in range(WVT // CH):
        off = sid * WVT + rblk * CH
        pltpu.sync_copy(wrA, wv_sh.at[pl.ds(off, CH)])
    for rblk in range(ZT_ // CH):
        off = sid * ZT_ + rblk * CH
        pltpu.sync_copy(wrA, z_sh.at[pl.ds(off, CH)])
    zoff = sid * ZT_ + (ZT_ // CH) * CH
    pltpu.sync_copy(wrA.at[pl.ds(0, ZT_ % CH)], z_sh.at[pl.ds(zoff, ZT_ % CH)])
    plsc.subcore_barrier()

    lane = lax.iota(jnp.int32, 16)
    hbase = cid * HH
    qbase = cid * DH

    ebase = sid * EPT
    NU = (NCHUNK - 1) // 2

    def stage_block(first_chunk):
        off = ebase + first_chunk * CH
        pltpu.async_copy(src_hbm.at[pl.ds(off, 2 * CH)], stg_s, isem)
        pltpu.async_copy(dst_hbm.at[pl.ds(off, 2 * CH)], stg_d, isem)

    def wait_stage():
        pltpu.make_async_copy(src_hbm.at[pl.ds(0, 2 * CH)], stg_s, isem).wait()
        pltpu.make_async_copy(dst_hbm.at[pl.ds(0, 2 * CH)], stg_d, isem).wait()

    def derive_pre(stg_off, s2, qd):
        for j in range(CH // 16):
            sv = stg_s[pl.ds(stg_off + 16 * j, 16)]
            dv = stg_d[pl.ds(stg_off + 16 * j, 16)]
            s2[pl.ds(16 * j, 16)] = sv + cid * N
            qd[pl.ds(16 * j, 16)] = dv

    def derive_post(qd, d2, d8):
        for j in range(CH // 16):
            dv = qd[pl.ds(16 * j, 16)]
            d2[pl.ds(16 * j, 16)] = dv >> 1
            d8[pl.ds(16 * j, 16)] = dv >> 3

    def issue_gathers(s2, qd, kkX, qqX, sem):
        pltpu.async_copy(kv_hbm.at[s2], kkX, sem)
        pltpu.async_copy(q_hbm.at[qd], qqX, sem)

    def wait_gathers(s2, qd, kkX, qqX, sem):
        pltpu.make_async_copy(kv_hbm.at[s2], kkX, sem).wait()
        pltpu.make_async_copy(q_hbm.at[qd], qqX, sem).wait()

    def issue_scatters(wrX, zrX, d2, d8, sem):
        pltpu.async_copy(wrX, wv_sh.at[d2], sem, add=True)
        pltpu.async_copy(zrX, z_sh.at[d8], sem, add=True)

    def wait_scatters(wrX, zrX, d2, d8, sem):
        pltpu.make_async_copy(wrX, wv_sh.at[d2], sem).wait()
        pltpu.make_async_copy(zrX, z_sh.at[d8], sem).wait()

    def compute(kkX, qqX, wrX, zrX, qd, psX):
        @plsc.parallel_loop(0, CH // 16)
        def group_body(g):
            dvec = qd[pl.ds(16 * g, 16)]
            pvec = psX[pl.ds(16 * g, 16)]
            for i in range(16):
                e = 16 * g + i
                d = dvec[i]
                ps = pvec[i]
                par = (d & 1) * DH
                opp = DH - par
                slot = (d & 7) * DK
                zacc = jnp.zeros((16,), jnp.float32)
                for h in range(HH):
                    kv = kkX[e, pl.ds(16 * h, 16)]
                    qv = qqX[e, pl.ds(qbase + 16 * h, 16)]
                    cs = plsc.cumsum(kv * qv)
                    sb = jnp.broadcast_to(cs[15], (16,))
                    es = jnp.exp(jnp.clip(sb, -5.0, 5.0))
                    wrX[e, pl.ds(par + 16 * h, 16)] = (
                        kkX[e, pl.ds(DH + 16 * h, 16)] * es)
                    wrX[e, pl.ds(opp + 16 * h, 16)] = zeros16
                    zacc = zacc + jnp.where(lane == hbase + h, es, 0.0)
                zrX[e, pl.ds(ps, 16)] = zeros16
                zrX[e, pl.ds(slot, 16)] = zacc
            psX[pl.ds(16 * g, 16)] = (dvec & 7) * DK

    pltpu.sync_copy(src_hbm.at[pl.ds(ebase, CH)], stg_s.at[pl.ds(0, CH)])
    pltpu.sync_copy(dst_hbm.at[pl.ds(ebase, CH)], stg_d.at[pl.ds(0, CH)])
    derive_pre(0, s2A, qdA)
    issue_gathers(s2A, qdA, kkA, qqA, gsA)
    stage_block(1)

    def pipe_body(u, carry):
        wait_stage()
        derive_pre(0, s2B, qdB)
        issue_gathers(s2B, qdB, kkB, qqB, gsB)
        wait_gathers(s2A, qdA, kkA, qqA, gsA)

        @pl.when(u > 0)
        def _():
            wait_scatters(wrA, zrA, d2A, d8A, ssA)

        derive_post(qdA, d2A, d8A)
        compute(kkA, qqA, wrA, zrA, qdA, psA)
        issue_scatters(wrA, zrA, d2A, d8A, ssA)

        derive_pre(CH, s2A, qdA)
        issue_gathers(s2A, qdA, kkA, qqA, gsA)

        @pl.when(u < NU - 1)
        def _():
            stage_block(2 * u + 3)

        wait_gathers(s2B, qdB, kkB, qqB, gsB)

        @pl.when(u > 0)
        def _():
            wait_scatters(wrB, zrB, d2B, d8B, ssB)

        derive_post(qdB, d2B, d8B)
        compute(kkB, qqB, wrB, zrB, qdB, psB)
        issue_scatters(wrB, zrB, d2B, d8B, ssB)
        return carry

    lax.fori_loop(0, NU, pipe_body, 0)

    wait_gathers(s2A, qdA, kkA, qqA, gsA)
    wait_scatters(wrA, zrA, d2A, d8A, ssA)
    derive_post(qdA, d2A, d8A)
    compute(kkA, qqA, wrA, zrA, qdA, psA)
    issue_scatters(wrA, zrA, d2A, d8A, ssA)
    wait_scatters(wrA, zrA, d2A, d8A, ssA)
    wait_scatters(wrB, zrB, d2B, d8B, ssB)

    plsc.subcore_barrier()
    woff = sid * WVT
    pltpu.sync_copy(wv_sh.at[pl.ds(woff, WVT)],
                    wv_out.at[cid, pl.ds(woff, WVT)])
    zoff2 = sid * ZT_
    pltpu.sync_copy(z_sh.at[pl.ds(zoff2, ZT_)],
                    z_out.at[cid, pl.ds(zoff2, ZT_)])


def kernel(x, edge_index, Wq, bq, Wk, Wv, Wo, bo, ln1_g, ln1_b, W1, b1, W2,
           b2, ln2_g, ln2_b):
    src = edge_index[0]
    dst = edge_index[1]
    for i in range(L):
        q, kv = _qkv_call(
            x, Wq[i], bq[i].reshape(1, D),
            Wk[i, :, :DH], Wk[i, :, DH:], Wv[i, :, :DH], Wv[i, :, DH:])
        wv_acc, z_acc = _edge_kernel(q, kv.reshape(NC * N, D), src, dst)
        wv_acc = wv_acc.reshape(NC, NP_, DH)
        z_acc = z_acc.reshape(NC, NP_, DK)
        x = _post_call(x, wv_acc, z_acc, Wo[i, :DH, :], Wo[i, DH:, :],
                       bo[i].reshape(1, D),
                       ln1_g[i].reshape(1, D), ln1_b[i].reshape(1, D),
                       W1[i], b1[i].reshape(1, DFF), W2[i],
                       b2[i].reshape(1, D), ln2_g[i].reshape(1, D),
                       ln2_b[i].reshape(1, D))
    return x
